# Initial kernel scaffold; baseline (speedup 1.0000x reference)
import jax
import jax.numpy as jnp
from jax.experimental import pallas as pl


def kernel(x, edge_attr, edge_index, extra_features, a_prelu, W_edge, W_eupd, W_node, b_node, We1, be1, We2, be2, We3, be3, Wg1, bg1, Wg2, bg2, Wg3, bg3, Wf, bf):
    raise NotImplementedError("write your pallas kernel here")


# speedup vs baseline: 1.7467x; 1.7467x over previous
import functools

import jax
import jax.numpy as jnp
from jax import lax
from jax.experimental import pallas as pl
from jax.experimental.pallas import tpu as pltpu
from jax.experimental.pallas import tpu_sc as plsc

N = 10000
E = 320000
D = 128
NC = 2
NS = 16
NW = NC * NS

GE = 256
NG = E // GE
KJ = GE // 128
ZCH = 200
NZC = N // ZCH


def _prelu(v, a):
    return jnp.where(v >= 0, v, a * v)


_MESH = plsc.VectorSubcoreMesh(core_axis_name="c", subcore_axis_name="s")


@functools.partial(
    pl.kernel,
    out_type=jax.ShapeDtypeStruct((E, D), jnp.float32),
    mesh=_MESH,
    scratch_types=[
        pltpu.VMEM((KJ, 128), jnp.int32),
        pltpu.VMEM((GE, D), jnp.float32),
        pltpu.SemaphoreType.DMA,
    ],
)
def _sc_gather_hbm(table_hbm, idx2_hbm, out_hbm, idx_v, buf_v, sem):
    wid = lax.axis_index("c") * NS + lax.axis_index("s")

    @pl.loop(0, pl.cdiv(NG, NW))
    def _groups(i):
        t = wid + i * NW

        @pl.when(t < NG)
        def _():
            pltpu.sync_copy(idx2_hbm.at[pl.ds(t * KJ, KJ)], idx_v)
            for j in range(KJ):
                pltpu.async_copy(
                    table_hbm.at[idx_v.at[j]],
                    buf_v.at[pl.ds(j * 128, 128)],
                    sem,
                ).wait()
            pltpu.sync_copy(buf_v, out_hbm.at[pl.ds(t * GE, GE)])


def _zero_and_scatter(zeros_hbm, h_hbm, dst2_hbm, table, idx_v, buf_v, sid):
    @pl.loop(0, pl.cdiv(NZC, NS))
    def _z(i):
        t = sid + i * NS

        @pl.when(t < NZC)
        def _():
            r0 = t * ZCH
            pltpu.sync_copy(zeros_hbm.at[pl.ds(r0, ZCH)],
                            table.at[pl.ds(r0, ZCH)])

    plsc.subcore_barrier()

    @pl.loop(0, pl.cdiv(NG, NS))
    def _groups(i):
        t = sid + i * NS

        @pl.when(t < NG)
        def _():
            pltpu.sync_copy(dst2_hbm.at[pl.ds(t * KJ, KJ)], idx_v)
            pltpu.sync_copy(h_hbm.at[pl.ds(t * GE, GE)], buf_v)
            for j in range(KJ):
                pltpu.sync_copy(
                    buf_v.at[pl.ds(j * 128, 128)],
                    table.at[idx_v.at[j]],
                    add=True,
                )

    plsc.subcore_barrier()


@functools.partial(
    pl.kernel,
    out_type=jax.ShapeDtypeStruct((E, D), jnp.float32),
    mesh=_MESH,
    scratch_types=[
        pltpu.VMEM((KJ, 128), jnp.int32),
        pltpu.VMEM((GE, D), jnp.float32),
        pltpu.VMEM_SHARED((N, D), jnp.float32),
    ],
)
def _sc_scatter_gather(h_hbm, dst2_hbm, src2_hbm, zeros_hbm, out_hbm,
                       idx_v, buf_v, table):
    cid = lax.axis_index("c")
    sid = lax.axis_index("s")
    wid = cid * NS + sid

    _zero_and_scatter(zeros_hbm, h_hbm, dst2_hbm, table, idx_v, buf_v, sid)

    @pl.loop(0, pl.cdiv(NG, NW))
    def _groups(i):
        t = wid + i * NW

        @pl.when(t < NG)
        def _():
            pltpu.sync_copy(src2_hbm.at[pl.ds(t * KJ, KJ)], idx_v)
            for j in range(KJ):
                pltpu.sync_copy(
                    table.at[idx_v.at[j]],
                    buf_v.at[pl.ds(j * 128, 128)],
                )
            pltpu.sync_copy(buf_v, out_hbm.at[pl.ds(t * GE, GE)])


@functools.partial(
    pl.kernel,
    out_type=jax.ShapeDtypeStruct((N, D), jnp.float32),
    mesh=_MESH,
    scratch_types=[
        pltpu.VMEM((KJ, 128), jnp.int32),
        pltpu.VMEM((GE, D), jnp.float32),
        pltpu.VMEM_SHARED((N, D), jnp.float32),
    ],
)
def _sc_segsum(h_hbm, dst2_hbm, zeros_hbm, out_hbm, idx_v, buf_v, table):
    cid = lax.axis_index("c")
    sid = lax.axis_index("s")
    wid = cid * NS + sid

    _zero_and_scatter(zeros_hbm, h_hbm, dst2_hbm, table, idx_v, buf_v, sid)

    @pl.loop(0, pl.cdiv(NZC, NW))
    def _chunks(i):
        t = wid + i * NW

        @pl.when(t < NZC)
        def _():
            r0 = t * ZCH
            pltpu.sync_copy(table.at[pl.ds(r0, ZCH)], buf_v.at[pl.ds(0, ZCH)])
            pltpu.sync_copy(buf_v.at[pl.ds(0, ZCH)], out_hbm.at[pl.ds(r0, ZCH)])


def _full(shape):
    return pl.BlockSpec(shape, lambda *_: tuple(0 for _ in shape))


def _node_pre_body(x_ref, wx_ref, wn_ref, xp_ref, xn_ref):
    xb = x_ref[...]
    xp_ref[...] = jnp.dot(xb, wx_ref[...], preferred_element_type=jnp.float32)
    xn_ref[...] = jnp.dot(xb, wn_ref[...], preferred_element_type=jnp.float32)


def _node_pre(x, WxT, Wn1T):
    bn = 1000
    return pl.pallas_call(
        _node_pre_body,
        grid=(N // bn,),
        in_specs=[
            pl.BlockSpec((bn, D), lambda i: (i, 0)),
            _full((D, D)),
            _full((D, D)),
        ],
        out_specs=[
            pl.BlockSpec((bn, D), lambda i: (i, 0)),
            pl.BlockSpec((bn, D), lambda i: (i, 0)),
        ],
        out_shape=[
            jax.ShapeDtypeStruct((N, D), jnp.float32),
            jax.ShapeDtypeStruct((N, D), jnp.float32),
        ],
    )(x, WxT, Wn1T)


def _h0_body(g0_ref, ea_ref, we_ref, a_ref, out_ref):
    a = a_ref[0, 0]
    y = g0_ref[...] + jnp.dot(ea_ref[...], we_ref[...],
                              preferred_element_type=jnp.float32)
    out_ref[...] = _prelu(y, a)


def _h0(g0, edge_attr, WeT, a2):
    be = 3200
    return pl.pallas_call(
        _h0_body,
        grid=(E // be,),
        in_specs=[
            pl.BlockSpec((be, D), lambda i: (i, 0)),
            pl.BlockSpec((be, 16), lambda i: (i, 0)),
            _full((16, D)),
            _full((1, 1)),
        ],
        out_specs=pl.BlockSpec((be, D), lambda i: (i, 0)),
        out_shape=jax.ShapeDtypeStruct((E, D), jnp.float32),
    )(g0, edge_attr, WeT, a2)


def _round_body(gp_ref, hp_ref, h0p_ref, wu_ref, a_ref, out_ref):
    a = a_ref[0, 0]
    wu = wu_ref[...]
    hb = hp_ref[...]
    hr = jnp.concatenate([hb[:, D:], hb[:, :D]], axis=1)
    m = gp_ref[...] - hr
    y = jnp.concatenate(
        [jnp.dot(m[:, :D], wu, preferred_element_type=jnp.float32),
         jnp.dot(m[:, D:], wu, preferred_element_type=jnp.float32)],
        axis=1,
    )
    out_ref[...] = _prelu(y + h0p_ref[...], a)


def _round_mm(g, h, h0p, WuT, a2):
    bp = 1600
    ep = E // 2
    gp = g.reshape(ep, 2 * D)
    hp = h.reshape(ep, 2 * D)
    out = pl.pallas_call(
        _round_body,
        grid=(ep // bp,),
        in_specs=[
            pl.BlockSpec((bp, 2 * D), lambda i: (i, 0)),
            pl.BlockSpec((bp, 2 * D), lambda i: (i, 0)),
            pl.BlockSpec((bp, 2 * D), lambda i: (i, 0)),
            _full((D, D)),
            _full((1, 1)),
        ],
        out_specs=pl.BlockSpec((bp, 2 * D), lambda i: (i, 0)),
        out_shape=jax.ShapeDtypeStruct((ep, 2 * D), jnp.float32),
    )(gp, hp, h0p, WuT, a2)
    return out.reshape(E, D)


def _node_out_body(xn_ref, ms_ref, wn2_ref, b_ref, a_ref, out_ref):
    a = a_ref[0, 0]
    hn = _prelu(
        xn_ref[...]
        + jnp.dot(ms_ref[...], wn2_ref[...], preferred_element_type=jnp.float32)
        + b_ref[...],
        a,
    )
    part = jnp.sum(hn, axis=0, keepdims=True)

    @pl.when(pl.program_id(0) == 0)
    def _():
        out_ref[...] = jnp.zeros_like(out_ref)

    out_ref[...] += part


def _node_out(xn, msum, Wn2T, b2, a2):
    bn = 1000
    return pl.pallas_call(
        _node_out_body,
        grid=(N // bn,),
        in_specs=[
            pl.BlockSpec((bn, D), lambda i: (i, 0)),
            pl.BlockSpec((bn, D), lambda i: (i, 0)),
            _full((D, D)),
            _full((1, D)),
            _full((1, 1)),
        ],
        out_specs=pl.BlockSpec((1, D), lambda i: (0, 0)),
        out_shape=jax.ShapeDtypeStruct((1, D), jnp.float32),
    )(xn, msum, Wn2T, b2, a2)


def _head_body(sol_ref, ex_ref, we1_ref, be1_ref, bd2_ref, be2_ref, bd3_ref,
               be3_ref, wg1_ref, bg1_ref, wg2_ref, bg2_ref, wg3_ref, bg3_ref,
               wf_ref, bf_ref, a_ref, out_ref):
    a = a_ref[0, 0]
    comb = jnp.concatenate([sol_ref[...], ex_ref[...]], axis=1)
    eo = _prelu(jnp.dot(comb, we1_ref[...], preferred_element_type=jnp.float32)
                + be1_ref[...], a)
    eo = _prelu(jnp.dot(eo, bd2_ref[...], preferred_element_type=jnp.float32)
                + be2_ref[...], a)
    eo = jnp.dot(eo, bd3_ref[...], preferred_element_type=jnp.float32) \
        + be3_ref[...]
    g = _prelu(jnp.dot(comb, wg1_ref[...], preferred_element_type=jnp.float32)
               + bg1_ref[...], a)
    g = _prelu(jnp.dot(g, wg2_ref[...], preferred_element_type=jnp.float32)
               + bg2_ref[...], a)
    gl = jnp.dot(g, wg3_ref[...], preferred_element_type=jnp.float32) \
        + bg3_ref[...]
    gl = gl - jnp.max(gl, axis=1, keepdims=True)
    egl = jnp.exp(gl)
    sm = egl / jnp.sum(egl, axis=1, keepdims=True)
    out = jnp.sum(eo * sm, axis=1, keepdims=True)
    out_ref[...] = out * wf_ref[...] + bf_ref[...]


def _head(solute, extra, We1fT, be1f, BD2, be2f, BD3, be3f,
          Wg1T, bg1f, Wg2T, bg2f, Wg3T, bg3f, Wf, bf2, a2):
    H8 = 8 * D
    args = (solute, extra, We1fT, be1f, BD2, be2f, BD3, be3f,
            Wg1T, bg1f, Wg2T, bg2f, Wg3T, bg3f, Wf, bf2, a2)
    return pl.pallas_call(
        _head_body,
        grid=(1,),
        in_specs=[_full(x.shape) for x in args],
        out_specs=_full((1, 1)),
        out_shape=jax.ShapeDtypeStruct((1, 1), jnp.float32),
    )(*args)


def kernel(x, edge_attr, edge_index, extra_features, a_prelu, W_edge, W_eupd,
           W_node, b_node, We1, be1, We2, be2, We3, be3, Wg1, bg1, Wg2, bg2,
           Wg3, bg3, Wf, bf):
    src2 = edge_index[0].astype(jnp.int32).reshape(E // 128, 128)
    dst2 = edge_index[1].astype(jnp.int32).reshape(E // 128, 128)
    a2 = jnp.reshape(a_prelu.astype(jnp.float32), (1, 1))
    zeros_nd = jnp.zeros((N, D), jnp.float32)

    WxT = W_edge[:, :D].T
    WeT = W_edge[:, D:].T
    WuT = W_eupd.T
    Wn1T = W_node[:, :D].T
    Wn2T = W_node[:, D:].T
    b2 = b_node.reshape(1, D)

    NE = We1.shape[0]
    We1fT = We1.reshape(NE * 128, D + 16).T
    be1f = be1.reshape(1, NE * 128)
    BD2 = jax.scipy.linalg.block_diag(*[We2[e].T for e in range(NE)])
    be2f = be2.reshape(1, NE * 128)
    BD3 = jax.scipy.linalg.block_diag(*[We3[e].T for e in range(NE)])
    be3f = be3.reshape(1, NE)
    bf2 = bf.reshape(1, 1)

    xp, xn = _node_pre(x, WxT, Wn1T)
    g0 = _sc_gather_hbm(xp, src2)
    h0 = _h0(g0, edge_attr, WeT, a2)
    h0p = h0.reshape(E // 2, 2 * D)

    h = h0
    for _ in range(3):
        g = _sc_scatter_gather(h, dst2, src2, zeros_nd)
        h = _round_mm(g, h, h0p, WuT, a2)

    msum = _sc_segsum(h, dst2, zeros_nd)
    solute = _node_out(xn, msum, Wn2T, b2, a2)

    return _head(solute, extra_features, We1fT, be1f, BD2, be2f, BD3, be3f,
                 Wg1.T, bg1.reshape(1, 128), Wg2.T, bg2.reshape(1, 128),
                 Wg3.T, bg3.reshape(1, NE), Wf, bf2, a2)


# speedup vs baseline: 3.2237x; 1.8456x over previous
import functools

import jax
import jax.numpy as jnp
from jax import lax
from jax.experimental import pallas as pl
from jax.experimental.pallas import tpu as pltpu
from jax.experimental.pallas import tpu_sc as plsc

N = 10000
E = 320000
D = 128
NC = 2
NS = 16
NW = NC * NS

GR = 128
NG = E // GR
NSL = -(-NG // NS)
NWL = -(-NG // NW)
ZCH = 80
NZC = N // ZCH


def _prelu(v, a):
    return jnp.where(v >= 0, v, a * v)


_MESH = plsc.VectorSubcoreMesh(core_axis_name="c", subcore_axis_name="s")

_SC_SCRATCH = [
    pltpu.VMEM((1, 128), jnp.int32),
    pltpu.VMEM((1, 128), jnp.int32),
    pltpu.VMEM((GR, D), jnp.float32),
    pltpu.VMEM((GR, D), jnp.float32),
    pltpu.SemaphoreType.DMA,
    pltpu.SemaphoreType.DMA,
    pltpu.SemaphoreType.DMA,
    pltpu.SemaphoreType.DMA,
    pltpu.SemaphoreType.DMA,
    pltpu.SemaphoreType.DMA,
]


def _fill_table(src_hbm, table, sid):
    @pl.loop(0, -(-NZC // NS))
    def _z(i):
        tz = sid + i * NS

        @pl.when(tz < NZC)
        def _():
            r0 = tz * ZCH
            pltpu.sync_copy(src_hbm.at[pl.ds(r0, ZCH)], table.at[pl.ds(r0, ZCH)])


@functools.partial(
    pl.kernel,
    out_type=jax.ShapeDtypeStruct((E, D), jnp.float32),
    mesh=_MESH,
    scratch_types=_SC_SCRATCH + [pltpu.VMEM_SHARED((N, D), jnp.float32)],
)
def _sc_round(h_hbm, idx2_hbm, zeros_hbm, out_hbm,
              ia0, ia1, da0, da1, sia0, sia1, sda0, sda1, sst0, sst1, table):
    cid = lax.axis_index("c")
    sid = lax.axis_index("s")
    wid = cid * NS + sid
    ias = (ia0, ia1)
    das = (da0, da1)
    sias = (sia0, sia1)
    sdas = (sda0, sda1)
    ssts = (sst0, sst1)

    for b in range(2):
        t = sid + b * NS
        pltpu.async_copy(idx2_hbm.at[pl.ds(t, 1)], ias[b], sias[b])
        pltpu.async_copy(h_hbm.at[pl.ds(t * GR, GR)], das[b], sdas[b])
    _fill_table(zeros_hbm, table, sid)
    plsc.subcore_barrier()

    @pl.loop(0, -(-NSL // 2))
    def _sc(i2):
        for b in range(2):
            i = i2 * 2 + b
            t = sid + i * NS

            @pl.when(t < NG)
            def _():
                pltpu.make_async_copy(
                    idx2_hbm.at[pl.ds(t, 1)], ias[b], sias[b]).wait()
                pltpu.make_async_copy(
                    h_hbm.at[pl.ds(t * GR, GR)], das[b], sdas[b]).wait()
                pltpu.sync_copy(das[b], table.at[ias[b].at[0]], add=True)
                tn = sid + (i + 2) * NS

                @pl.when(tn < NG)
                def _():
                    pltpu.async_copy(idx2_hbm.at[pl.ds(tn, 1)], ias[b], sias[b])
                    pltpu.async_copy(
                        h_hbm.at[pl.ds(tn * GR, GR)], das[b], sdas[b])

    plsc.subcore_barrier()

    for b in range(2):
        t = wid + b * NW
        pltpu.async_copy(idx2_hbm.at[pl.ds(t, 1)], ias[b], sias[b])

    @pl.loop(0, -(-NWL // 2))
    def _ga(i2):
        for b in range(2):
            i = i2 * 2 + b
            t = wid + i * NW

            @pl.when(t < NG)
            def _():
                @pl.when(i >= 2)
                def _():
                    tprev = wid + (i - 2) * NW
                    pltpu.make_async_copy(
                        das[b], out_hbm.at[pl.ds(tprev * GR, GR)],
                        ssts[b]).wait()

                pltpu.make_async_copy(
                    idx2_hbm.at[pl.ds(t, 1)], ias[b], sias[b]).wait()
                pltpu.sync_copy(table.at[ias[b].at[0]], das[b])
                pltpu.async_copy(das[b], out_hbm.at[pl.ds(t * GR, GR)], ssts[b])
                tn = wid + (i + 2) * NW

                @pl.when(tn < NG)
                def _():
                    pltpu.async_copy(idx2_hbm.at[pl.ds(tn, 1)], ias[b], sias[b])

    imax = (NG - 1 - wid) // NW
    for b in range(2):
        ib = imax - lax.rem(imax - b, 2)
        tb = wid + ib * NW
        pltpu.make_async_copy(das[b], out_hbm.at[pl.ds(tb * GR, GR)],
                              ssts[b]).wait()


@functools.partial(
    pl.kernel,
    out_type=jax.ShapeDtypeStruct((E, D), jnp.float32),
    mesh=_MESH,
    scratch_types=_SC_SCRATCH + [pltpu.VMEM_SHARED((N, D), jnp.float32)],
)
def _sc_gather_spm(xp_hbm, idx2_hbm, out_hbm,
                   ia0, ia1, da0, da1, sia0, sia1, sda0, sda1, sst0, sst1,
                   table):
    cid = lax.axis_index("c")
    sid = lax.axis_index("s")
    wid = cid * NS + sid
    ias = (ia0, ia1)
    das = (da0, da1)
    sias = (sia0, sia1)
    ssts = (sst0, sst1)

    _fill_table(xp_hbm, table, sid)
    plsc.subcore_barrier()

    for b in range(2):
        t = wid + b * NW
        pltpu.async_copy(idx2_hbm.at[pl.ds(t, 1)], ias[b], sias[b])

    @pl.loop(0, -(-NWL // 2))
    def _ga(i2):
        for b in range(2):
            i = i2 * 2 + b
            t = wid + i * NW

            @pl.when(t < NG)
            def _():
                @pl.when(i >= 2)
                def _():
                    tprev = wid + (i - 2) * NW
                    pltpu.make_async_copy(
                        das[b], out_hbm.at[pl.ds(tprev * GR, GR)],
                        ssts[b]).wait()

                pltpu.make_async_copy(
                    idx2_hbm.at[pl.ds(t, 1)], ias[b], sias[b]).wait()
                pltpu.sync_copy(table.at[ias[b].at[0]], das[b])
                pltpu.async_copy(das[b], out_hbm.at[pl.ds(t * GR, GR)], ssts[b])
                tn = wid + (i + 2) * NW

                @pl.when(tn < NG)
                def _():
                    pltpu.async_copy(idx2_hbm.at[pl.ds(tn, 1)], ias[b], sias[b])

    imax = (NG - 1 - wid) // NW
    for b in range(2):
        ib = imax - lax.rem(imax - b, 2)
        tb = wid + ib * NW
        pltpu.make_async_copy(das[b], out_hbm.at[pl.ds(tb * GR, GR)],
                              ssts[b]).wait()


@functools.partial(
    pl.kernel,
    out_type=jax.ShapeDtypeStruct((E, D), jnp.float32),
    mesh=_MESH,
    scratch_types=_SC_SCRATCH,
)
def _sc_swap_pairs(h_hbm, rev2_hbm, out_hbm,
                   ia0, ia1, da0, da1, sia0, sia1, sda0, sda1, sst0, sst1):
    cid = lax.axis_index("c")
    sid = lax.axis_index("s")
    wid = cid * NS + sid
    ias = (ia0, ia1)
    das = (da0, da1)
    sias = (sia0, sia1)
    sdas = (sda0, sda1)

    for b in range(2):
        t = wid + b * NW
        pltpu.async_copy(rev2_hbm.at[pl.ds(t, 1)], ias[b], sias[b])
        pltpu.async_copy(h_hbm.at[pl.ds(t * GR, GR)], das[b], sdas[b])

    @pl.loop(0, -(-NWL // 2))
    def _sw(i2):
        for b in range(2):
            i = i2 * 2 + b
            t = wid + i * NW

            @pl.when(t < NG)
            def _():
                pltpu.make_async_copy(
                    rev2_hbm.at[pl.ds(t, 1)], ias[b], sias[b]).wait()
                pltpu.make_async_copy(
                    h_hbm.at[pl.ds(t * GR, GR)], das[b], sdas[b]).wait()
                pltpu.sync_copy(das[b], out_hbm.at[ias[b].at[0]])
                tn = wid + (i + 2) * NW

                @pl.when(tn < NG)
                def _():
                    pltpu.async_copy(rev2_hbm.at[pl.ds(tn, 1)], ias[b], sias[b])
                    pltpu.async_copy(
                        h_hbm.at[pl.ds(tn * GR, GR)], das[b], sdas[b])


@functools.partial(
    pl.kernel,
    out_type=jax.ShapeDtypeStruct((N, D), jnp.float32),
    mesh=_MESH,
    scratch_types=_SC_SCRATCH + [pltpu.VMEM_SHARED((N, D), jnp.float32)],
)
def _sc_segsum(h_hbm, idx2_hbm, zeros_hbm, out_hbm,
               ia0, ia1, da0, da1, sia0, sia1, sda0, sda1, sst0, sst1, table):
    cid = lax.axis_index("c")
    sid = lax.axis_index("s")
    wid = cid * NS + sid
    ias = (ia0, ia1)
    das = (da0, da1)
    sias = (sia0, sia1)
    sdas = (sda0, sda1)

    for b in range(2):
        t = sid + b * NS
        pltpu.async_copy(idx2_hbm.at[pl.ds(t, 1)], ias[b], sias[b])
        pltpu.async_copy(h_hbm.at[pl.ds(t * GR, GR)], das[b], sdas[b])
    _fill_table(zeros_hbm, table, sid)
    plsc.subcore_barrier()

    @pl.loop(0, -(-NSL // 2))
    def _sc(i2):
        for b in range(2):
            i = i2 * 2 + b
            t = sid + i * NS

            @pl.when(t < NG)
            def _():
                pltpu.make_async_copy(
                    idx2_hbm.at[pl.ds(t, 1)], ias[b], sias[b]).wait()
                pltpu.make_async_copy(
                    h_hbm.at[pl.ds(t * GR, GR)], das[b], sdas[b]).wait()
                pltpu.sync_copy(das[b], table.at[ias[b].at[0]], add=True)
                tn = sid + (i + 2) * NS

                @pl.when(tn < NG)
                def _():
                    pltpu.async_copy(idx2_hbm.at[pl.ds(tn, 1)], ias[b], sias[b])
                    pltpu.async_copy(
                        h_hbm.at[pl.ds(tn * GR, GR)], das[b], sdas[b])

    plsc.subcore_barrier()

    @pl.loop(0, -(-NZC // NW))
    def _chunks(i):
        t = wid + i * NW

        @pl.when(t < NZC)
        def _():
            r0 = t * ZCH
            pltpu.sync_copy(table.at[pl.ds(r0, ZCH)], da0.at[pl.ds(0, ZCH)])
            pltpu.sync_copy(da0.at[pl.ds(0, ZCH)], out_hbm.at[pl.ds(r0, ZCH)])


def _full(shape):
    return pl.BlockSpec(shape, lambda *_: tuple(0 for _ in shape))


def _node_pre_body(x_ref, wx_ref, wn_ref, xp_ref, xn_ref):
    xb = x_ref[...]
    xp_ref[...] = jnp.dot(xb, wx_ref[...], preferred_element_type=jnp.float32)
    xn_ref[...] = jnp.dot(xb, wn_ref[...], preferred_element_type=jnp.float32)


def _node_pre(x, WxT, Wn1T):
    bn = 1000
    return pl.pallas_call(
        _node_pre_body,
        grid=(N // bn,),
        in_specs=[
            pl.BlockSpec((bn, D), lambda i: (i, 0)),
            _full((D, D)),
            _full((D, D)),
        ],
        out_specs=[
            pl.BlockSpec((bn, D), lambda i: (i, 0)),
            pl.BlockSpec((bn, D), lambda i: (i, 0)),
        ],
        out_shape=[
            jax.ShapeDtypeStruct((N, D), jnp.float32),
            jax.ShapeDtypeStruct((N, D), jnp.float32),
        ],
    )(x, WxT, Wn1T)


def _h0_body(g0_ref, ea_ref, we_ref, a_ref, out_ref):
    a = a_ref[0, 0]
    y = g0_ref[...] + jnp.dot(ea_ref[...], we_ref[...],
                              preferred_element_type=jnp.float32)
    out_ref[...] = _prelu(y, a)


def _h0(g0, edge_attr, WeT, a2):
    be = 3200
    return pl.pallas_call(
        _h0_body,
        grid=(E // be,),
        in_specs=[
            pl.BlockSpec((be, D), lambda i: (i, 0)),
            pl.BlockSpec((be, 16), lambda i: (i, 0)),
            _full((16, D)),
            _full((1, 1)),
        ],
        out_specs=pl.BlockSpec((be, D), lambda i: (i, 0)),
        out_shape=jax.ShapeDtypeStruct((E, D), jnp.float32),
    )(g0, edge_attr, WeT, a2)


def _round_body(g_ref, h_ref, h0_ref, wu_ref, a_ref, out_ref):
    a = a_ref[0, 0]
    m = g_ref[...] - h_ref[...]
    y = jnp.dot(m, wu_ref[...], preferred_element_type=jnp.float32)
    out_ref[...] = _prelu(y + h0_ref[...], a)


def _round_tc(g, h, h0sel, WuT, a2):
    be = 3200
    return pl.pallas_call(
        _round_body,
        grid=(E // be,),
        in_specs=[
            pl.BlockSpec((be, D), lambda i: (i, 0)),
            pl.BlockSpec((be, D), lambda i: (i, 0)),
            pl.BlockSpec((be, D), lambda i: (i, 0)),
            _full((D, D)),
            _full((1, 1)),
        ],
        out_specs=pl.BlockSpec((be, D), lambda i: (i, 0)),
        out_shape=jax.ShapeDtypeStruct((E, D), jnp.float32),
    )(g, h, h0sel, WuT, a2)


def _node_out_body(xn_ref, ms_ref, wn2_ref, b_ref, a_ref, out_ref):
    a = a_ref[0, 0]
    hn = _prelu(
        xn_ref[...]
        + jnp.dot(ms_ref[...], wn2_ref[...], preferred_element_type=jnp.float32)
        + b_ref[...],
        a,
    )
    part = jnp.sum(hn, axis=0, keepdims=True)

    @pl.when(pl.program_id(0) == 0)
    def _():
        out_ref[...] = jnp.zeros_like(out_ref)

    out_ref[...] += part


def _node_out(xn, msum, Wn2T, b2, a2):
    bn = 1000
    return pl.pallas_call(
        _node_out_body,
        grid=(N // bn,),
        in_specs=[
            pl.BlockSpec((bn, D), lambda i: (i, 0)),
            pl.BlockSpec((bn, D), lambda i: (i, 0)),
            _full((D, D)),
            _full((1, D)),
            _full((1, 1)),
        ],
        out_specs=pl.BlockSpec((1, D), lambda i: (0, 0)),
        out_shape=jax.ShapeDtypeStruct((1, D), jnp.float32),
    )(xn, msum, Wn2T, b2, a2)


def _head_body(sol_ref, ex_ref, we1_ref, be1_ref, bd2_ref, be2_ref, bd3_ref,
               be3_ref, wg1_ref, bg1_ref, wg2_ref, bg2_ref, wg3_ref, bg3_ref,
               wf_ref, bf_ref, a_ref, out_ref):
    a = a_ref[0, 0]
    comb = jnp.concatenate([sol_ref[...], ex_ref[...]], axis=1)
    eo = _prelu(jnp.dot(comb, we1_ref[...], preferred_element_type=jnp.float32)
                + be1_ref[...], a)
    eo = _prelu(jnp.dot(eo, bd2_ref[...], preferred_element_type=jnp.float32)
                + be2_ref[...], a)
    eo = jnp.dot(eo, bd3_ref[...], preferred_element_type=jnp.float32) \
        + be3_ref[...]
    g = _prelu(jnp.dot(comb, wg1_ref[...], preferred_element_type=jnp.float32)
               + bg1_ref[...], a)
    g = _prelu(jnp.dot(g, wg2_ref[...], preferred_element_type=jnp.float32)
               + bg2_ref[...], a)
    gl = jnp.dot(g, wg3_ref[...], preferred_element_type=jnp.float32) \
        + bg3_ref[...]
    gl = gl - jnp.max(gl, axis=1, keepdims=True)
    egl = jnp.exp(gl)
    sm = egl / jnp.sum(egl, axis=1, keepdims=True)
    out = jnp.sum(eo * sm, axis=1, keepdims=True)
    out_ref[...] = out * wf_ref[...] + bf_ref[...]


def _head(solute, extra, We1fT, be1f, BD2, be2f, BD3, be3f,
          Wg1T, bg1f, Wg2T, bg2f, Wg3T, bg3f, Wf, bf2, a2):
    args = (solute, extra, We1fT, be1f, BD2, be2f, BD3, be3f,
            Wg1T, bg1f, Wg2T, bg2f, Wg3T, bg3f, Wf, bf2, a2)
    return pl.pallas_call(
        _head_body,
        grid=(1,),
        in_specs=[_full(x.shape) for x in args],
        out_specs=_full((1, 1)),
        out_shape=jax.ShapeDtypeStruct((1, 1), jnp.float32),
    )(*args)


def kernel(x, edge_attr, edge_index, extra_features, a_prelu, W_edge, W_eupd,
           W_node, b_node, We1, be1, We2, be2, We3, be3, Wg1, bg1, Wg2, bg2,
           Wg3, bg3, Wf, bf):
    src2 = edge_index[0].astype(jnp.int32).reshape(E // GR, GR)
    dst2 = edge_index[1].astype(jnp.int32).reshape(E // GR, GR)
    rev2 = (jnp.arange(E, dtype=jnp.int32) ^ 1).reshape(E // GR, GR)
    a2 = jnp.reshape(a_prelu.astype(jnp.float32), (1, 1))
    zeros_nd = jnp.zeros((N, D), jnp.float32)

    WxT = W_edge[:, :D].T
    WeT = W_edge[:, D:].T
    WuT = W_eupd.T
    Wn1T = W_node[:, :D].T
    Wn2T = W_node[:, D:].T
    b2 = b_node.reshape(1, D)

    NE = We1.shape[0]
    We1fT = We1.reshape(NE * 128, D + 16).T
    be1f = be1.reshape(1, NE * 128)
    BD2 = jax.scipy.linalg.block_diag(*[We2[e].T for e in range(NE)])
    be2f = be2.reshape(1, NE * 128)
    BD3 = jax.scipy.linalg.block_diag(*[We3[e].T for e in range(NE)])
    be3f = be3.reshape(1, NE)
    bf2 = bf.reshape(1, 1)

    xp, xn = _node_pre(x, WxT, Wn1T)
    g0 = _sc_gather_spm(xp, src2)
    h0 = _h0(g0, edge_attr, WeT, a2)
    H0 = _sc_swap_pairs(h0, rev2)

    g = _sc_round(h0, dst2, zeros_nd)
    H1 = _round_tc(g, h0, H0, WuT, a2)
    g = _sc_round(H1, src2, zeros_nd)
    h2 = _round_tc(g, H1, h0, WuT, a2)
    g = _sc_round(h2, dst2, zeros_nd)
    H3 = _round_tc(g, h2, H0, WuT, a2)

    msum = _sc_segsum(H3, src2, zeros_nd)
    solute = _node_out(xn, msum, Wn2T, b2, a2)

    return _head(solute, extra_features, We1fT, be1f, BD2, be2f, BD3, be3f,
                 Wg1.T, bg1.reshape(1, 128), Wg2.T, bg2.reshape(1, 128),
                 Wg3.T, bg3.reshape(1, NE), Wf, bf2, a2)


# speedup vs baseline: 3.3518x; 1.0397x over previous
import functools

import jax
import jax.numpy as jnp
from jax import lax
from jax.experimental import pallas as pl
from jax.experimental.pallas import tpu as pltpu
from jax.experimental.pallas import tpu_sc as plsc

N = 10000
E = 320000
D = 128
NC = 2
NS = 16
NW = NC * NS

GR = 128
NG = E // GR
NSL = -(-NG // NS)
NWL = -(-NG // NW)
ZCH = 80
NZC = N // ZCH


def _prelu(v, a):
    return jnp.where(v >= 0, v, a * v)


_MESH = plsc.VectorSubcoreMesh(core_axis_name="c", subcore_axis_name="s")

_SC_SCRATCH = [
    pltpu.VMEM((1, 128), jnp.int32),
    pltpu.VMEM((1, 128), jnp.int32),
    pltpu.VMEM((GR, D), jnp.float32),
    pltpu.VMEM((GR, D), jnp.float32),
    pltpu.SemaphoreType.DMA,
    pltpu.SemaphoreType.DMA,
    pltpu.SemaphoreType.DMA,
    pltpu.SemaphoreType.DMA,
    pltpu.SemaphoreType.DMA,
    pltpu.SemaphoreType.DMA,
]


def _fill_table(src_hbm, table, sid):
    @pl.loop(0, -(-NZC // NS))
    def _z(i):
        tz = sid + i * NS

        @pl.when(tz < NZC)
        def _():
            r0 = tz * ZCH
            pltpu.sync_copy(src_hbm.at[pl.ds(r0, ZCH)], table.at[pl.ds(r0, ZCH)])


@functools.partial(
    pl.kernel,
    out_type=jax.ShapeDtypeStruct((E, D), jnp.float32),
    mesh=_MESH,
    scratch_types=_SC_SCRATCH + [pltpu.VMEM_SHARED((N, D), jnp.float32)],
)
def _sc_round(h_hbm, idx2_hbm, zeros_hbm, out_hbm,
              ia0, ia1, da0, da1, sia0, sia1, sda0, sda1, sst0, sst1, table):
    cid = lax.axis_index("c")
    sid = lax.axis_index("s")
    wid = cid * NS + sid
    ias = (ia0, ia1)
    das = (da0, da1)
    sias = (sia0, sia1)
    sdas = (sda0, sda1)
    ssts = (sst0, sst1)

    for b in range(2):
        t = sid + b * NS
        pltpu.async_copy(idx2_hbm.at[pl.ds(t, 1)], ias[b], sias[b])
        pltpu.async_copy(h_hbm.at[pl.ds(t * GR, GR)], das[b], sdas[b])
    _fill_table(zeros_hbm, table, sid)
    plsc.subcore_barrier()

    @pl.loop(0, -(-NSL // 2))
    def _sc(i2):
        for b in range(2):
            i = i2 * 2 + b
            t = sid + i * NS

            @pl.when(t < NG)
            def _():
                pltpu.make_async_copy(
                    idx2_hbm.at[pl.ds(t, 1)], ias[b], sias[b]).wait()
                pltpu.make_async_copy(
                    h_hbm.at[pl.ds(t * GR, GR)], das[b], sdas[b]).wait()
                pltpu.sync_copy(das[b], table.at[ias[b].at[0]], add=True)
                tn = sid + (i + 2) * NS

                @pl.when(tn < NG)
                def _():
                    pltpu.async_copy(idx2_hbm.at[pl.ds(tn, 1)], ias[b], sias[b])
                    pltpu.async_copy(
                        h_hbm.at[pl.ds(tn * GR, GR)], das[b], sdas[b])

    plsc.subcore_barrier()

    for b in range(2):
        t = wid + b * NW
        pltpu.async_copy(idx2_hbm.at[pl.ds(t, 1)], ias[b], sias[b])

    @pl.loop(0, -(-NWL // 2))
    def _ga(i2):
        for b in range(2):
            i = i2 * 2 + b
            t = wid + i * NW

            @pl.when(t < NG)
            def _():
                @pl.when(i >= 2)
                def _():
                    tprev = wid + (i - 2) * NW
                    pltpu.make_async_copy(
                        das[b], out_hbm.at[pl.ds(tprev * GR, GR)],
                        ssts[b]).wait()

                pltpu.make_async_copy(
                    idx2_hbm.at[pl.ds(t, 1)], ias[b], sias[b]).wait()
                pltpu.sync_copy(table.at[ias[b].at[0]], das[b])
                pltpu.async_copy(das[b], out_hbm.at[pl.ds(t * GR, GR)], ssts[b])
                tn = wid + (i + 2) * NW

                @pl.when(tn < NG)
                def _():
                    pltpu.async_copy(idx2_hbm.at[pl.ds(tn, 1)], ias[b], sias[b])

    imax = (NG - 1 - wid) // NW
    for b in range(2):
        ib = imax - lax.rem(imax - b, 2)
        tb = wid + ib * NW
        pltpu.make_async_copy(das[b], out_hbm.at[pl.ds(tb * GR, GR)],
                              ssts[b]).wait()


@functools.partial(
    pl.kernel,
    out_type=jax.ShapeDtypeStruct((E, D), jnp.float32),
    mesh=_MESH,
    scratch_types=_SC_SCRATCH + [pltpu.VMEM_SHARED((N, D), jnp.float32)],
)
def _sc_gather_spm(xp_hbm, idx2_hbm, out_hbm,
                   ia0, ia1, da0, da1, sia0, sia1, sda0, sda1, sst0, sst1,
                   table):
    cid = lax.axis_index("c")
    sid = lax.axis_index("s")
    wid = cid * NS + sid
    ias = (ia0, ia1)
    das = (da0, da1)
    sias = (sia0, sia1)
    ssts = (sst0, sst1)

    _fill_table(xp_hbm, table, sid)
    plsc.subcore_barrier()

    for b in range(2):
        t = wid + b * NW
        pltpu.async_copy(idx2_hbm.at[pl.ds(t, 1)], ias[b], sias[b])

    @pl.loop(0, -(-NWL // 2))
    def _ga(i2):
        for b in range(2):
            i = i2 * 2 + b
            t = wid + i * NW

            @pl.when(t < NG)
            def _():
                @pl.when(i >= 2)
                def _():
                    tprev = wid + (i - 2) * NW
                    pltpu.make_async_copy(
                        das[b], out_hbm.at[pl.ds(tprev * GR, GR)],
                        ssts[b]).wait()

                pltpu.make_async_copy(
                    idx2_hbm.at[pl.ds(t, 1)], ias[b], sias[b]).wait()
                pltpu.sync_copy(table.at[ias[b].at[0]], das[b])
                pltpu.async_copy(das[b], out_hbm.at[pl.ds(t * GR, GR)], ssts[b])
                tn = wid + (i + 2) * NW

                @pl.when(tn < NG)
                def _():
                    pltpu.async_copy(idx2_hbm.at[pl.ds(tn, 1)], ias[b], sias[b])

    imax = (NG - 1 - wid) // NW
    for b in range(2):
        ib = imax - lax.rem(imax - b, 2)
        tb = wid + ib * NW
        pltpu.make_async_copy(das[b], out_hbm.at[pl.ds(tb * GR, GR)],
                              ssts[b]).wait()


@functools.partial(
    pl.kernel,
    out_type=jax.ShapeDtypeStruct((E, D), jnp.float32),
    mesh=_MESH,
    scratch_types=_SC_SCRATCH,
)
def _sc_swap_pairs(h_hbm, rev2_hbm, out_hbm,
                   ia0, ia1, da0, da1, sia0, sia1, sda0, sda1, sst0, sst1):
    cid = lax.axis_index("c")
    sid = lax.axis_index("s")
    wid = cid * NS + sid
    ias = (ia0, ia1)
    das = (da0, da1)
    sias = (sia0, sia1)
    sdas = (sda0, sda1)

    for b in range(2):
        t = wid + b * NW
        pltpu.async_copy(rev2_hbm.at[pl.ds(t, 1)], ias[b], sias[b])
        pltpu.async_copy(h_hbm.at[pl.ds(t * GR, GR)], das[b], sdas[b])

    @pl.loop(0, -(-NWL // 2))
    def _sw(i2):
        for b in range(2):
            i = i2 * 2 + b
            t = wid + i * NW

            @pl.when(t < NG)
            def _():
                pltpu.make_async_copy(
                    rev2_hbm.at[pl.ds(t, 1)], ias[b], sias[b]).wait()
                pltpu.make_async_copy(
                    h_hbm.at[pl.ds(t * GR, GR)], das[b], sdas[b]).wait()
                pltpu.sync_copy(das[b], out_hbm.at[ias[b].at[0]])
                tn = wid + (i + 2) * NW

                @pl.when(tn < NG)
                def _():
                    pltpu.async_copy(rev2_hbm.at[pl.ds(tn, 1)], ias[b], sias[b])
                    pltpu.async_copy(
                        h_hbm.at[pl.ds(tn * GR, GR)], das[b], sdas[b])


@functools.partial(
    pl.kernel,
    out_type=[jax.ShapeDtypeStruct((N, D), jnp.float32),
              jax.ShapeDtypeStruct((N, D), jnp.float32)],
    mesh=_MESH,
    scratch_types=_SC_SCRATCH + [pltpu.VMEM_SHARED((N, D), jnp.float32)],
)
def _sc_segsum(h_hbm, idx2_hbm, zeros_hbm, out0_hbm, out1_hbm,
               ia0, ia1, da0, da1, sia0, sia1, sda0, sda1, sst0, sst1, table):
    cid = lax.axis_index("c")
    sid = lax.axis_index("s")
    ias = (ia0, ia1)
    das = (da0, da1)
    sias = (sia0, sia1)
    sdas = (sda0, sda1)

    for b in range(2):
        t = sid + (2 * b + cid) * NS
        pltpu.async_copy(idx2_hbm.at[pl.ds(t, 1)], ias[b], sias[b])
        pltpu.async_copy(h_hbm.at[pl.ds(t * GR, GR)], das[b], sdas[b])
    _fill_table(zeros_hbm, table, sid)
    plsc.subcore_barrier()

    NHL = -(-NSL // 2)
    @pl.loop(0, -(-NHL // 2))
    def _sc(j2):
        for b in range(2):
            j = j2 * 2 + b
            t = sid + (2 * j + cid) * NS

            @pl.when(t < NG)
            def _():
                pltpu.make_async_copy(
                    idx2_hbm.at[pl.ds(t, 1)], ias[b], sias[b]).wait()
                pltpu.make_async_copy(
                    h_hbm.at[pl.ds(t * GR, GR)], das[b], sdas[b]).wait()
                pltpu.sync_copy(das[b], table.at[ias[b].at[0]], add=True)
                tn = sid + (2 * (j + 2) + cid) * NS

                @pl.when(tn < NG)
                def _():
                    pltpu.async_copy(idx2_hbm.at[pl.ds(tn, 1)], ias[b], sias[b])
                    pltpu.async_copy(
                        h_hbm.at[pl.ds(tn * GR, GR)], das[b], sdas[b])

    plsc.subcore_barrier()

    @pl.loop(0, -(-NZC // NS))
    def _chunks(i):
        t = sid + i * NS

        @pl.when(t < NZC)
        def _():
            r0 = t * ZCH
            pltpu.sync_copy(table.at[pl.ds(r0, ZCH)], da0.at[pl.ds(0, ZCH)])

            @pl.when(cid == 0)
            def _():
                pltpu.sync_copy(da0.at[pl.ds(0, ZCH)],
                                out0_hbm.at[pl.ds(r0, ZCH)])

            @pl.when(cid == 1)
            def _():
                pltpu.sync_copy(da0.at[pl.ds(0, ZCH)],
                                out1_hbm.at[pl.ds(r0, ZCH)])


def _full(shape):
    return pl.BlockSpec(shape, lambda *_: tuple(0 for _ in shape))


def _node_pre_body(x_ref, wx_ref, wn_ref, xp_ref, xn_ref):
    xb = x_ref[...]
    xp_ref[...] = jnp.dot(xb, wx_ref[...], preferred_element_type=jnp.float32)
    xn_ref[...] = jnp.dot(xb, wn_ref[...], preferred_element_type=jnp.float32)


def _node_pre(x, WxT, Wn1T):
    bn = 1000
    return pl.pallas_call(
        _node_pre_body,
        grid=(N // bn,),
        in_specs=[
            pl.BlockSpec((bn, D), lambda i: (i, 0)),
            _full((D, D)),
            _full((D, D)),
        ],
        out_specs=[
            pl.BlockSpec((bn, D), lambda i: (i, 0)),
            pl.BlockSpec((bn, D), lambda i: (i, 0)),
        ],
        out_shape=[
            jax.ShapeDtypeStruct((N, D), jnp.float32),
            jax.ShapeDtypeStruct((N, D), jnp.float32),
        ],
    )(x, WxT, Wn1T)


def _h0_body(g0_ref, ea_ref, we_ref, a_ref, out_ref):
    a = a_ref[0, 0]
    y = g0_ref[...] + jnp.dot(ea_ref[...], we_ref[...],
                              preferred_element_type=jnp.float32)
    out_ref[...] = _prelu(y, a)


def _h0(g0, edge_attr, WeT, a2):
    be = 3200
    return pl.pallas_call(
        _h0_body,
        grid=(E // be,),
        in_specs=[
            pl.BlockSpec((be, D), lambda i: (i, 0)),
            pl.BlockSpec((be, 16), lambda i: (i, 0)),
            _full((16, D)),
            _full((1, 1)),
        ],
        out_specs=pl.BlockSpec((be, D), lambda i: (i, 0)),
        out_shape=jax.ShapeDtypeStruct((E, D), jnp.float32),
    )(g0, edge_attr, WeT, a2)


def _round_body(g_ref, h_ref, h0_ref, wu_ref, a_ref, out_ref):
    a = a_ref[0, 0]
    m = g_ref[...] - h_ref[...]
    y = jnp.dot(m, wu_ref[...], preferred_element_type=jnp.float32)
    out_ref[...] = _prelu(y + h0_ref[...], a)


def _round_tc(g, h, h0sel, WuT, a2):
    be = 3200
    return pl.pallas_call(
        _round_body,
        grid=(E // be,),
        in_specs=[
            pl.BlockSpec((be, D), lambda i: (i, 0)),
            pl.BlockSpec((be, D), lambda i: (i, 0)),
            pl.BlockSpec((be, D), lambda i: (i, 0)),
            _full((D, D)),
            _full((1, 1)),
        ],
        out_specs=pl.BlockSpec((be, D), lambda i: (i, 0)),
        out_shape=jax.ShapeDtypeStruct((E, D), jnp.float32),
    )(g, h, h0sel, WuT, a2)


def _node_out_body(xn_ref, ms0_ref, ms1_ref, wn2_ref, b_ref, a_ref, out_ref):
    a = a_ref[0, 0]
    ms = ms0_ref[...] + ms1_ref[...]
    hn = _prelu(
        xn_ref[...]
        + jnp.dot(ms, wn2_ref[...], preferred_element_type=jnp.float32)
        + b_ref[...],
        a,
    )
    part = jnp.sum(hn, axis=0, keepdims=True)

    @pl.when(pl.program_id(0) == 0)
    def _():
        out_ref[...] = jnp.zeros_like(out_ref)

    out_ref[...] += part


def _node_out(xn, ms0, ms1, Wn2T, b2, a2):
    bn = 1000
    return pl.pallas_call(
        _node_out_body,
        grid=(N // bn,),
        in_specs=[
            pl.BlockSpec((bn, D), lambda i: (i, 0)),
            pl.BlockSpec((bn, D), lambda i: (i, 0)),
            pl.BlockSpec((bn, D), lambda i: (i, 0)),
            _full((D, D)),
            _full((1, D)),
            _full((1, 1)),
        ],
        out_specs=pl.BlockSpec((1, D), lambda i: (0, 0)),
        out_shape=jax.ShapeDtypeStruct((1, D), jnp.float32),
    )(xn, ms0, ms1, Wn2T, b2, a2)


def _head_body(sol_ref, ex_ref, we1_ref, be1_ref, bd2_ref, be2_ref, bd3_ref,
               be3_ref, wg1_ref, bg1_ref, wg2_ref, bg2_ref, wg3_ref, bg3_ref,
               wf_ref, bf_ref, a_ref, out_ref):
    a = a_ref[0, 0]
    comb = jnp.concatenate([sol_ref[...], ex_ref[...]], axis=1)
    eo = _prelu(jnp.dot(comb, we1_ref[...], preferred_element_type=jnp.float32)
                + be1_ref[...], a)
    eo = _prelu(jnp.dot(eo, bd2_ref[...], preferred_element_type=jnp.float32)
                + be2_ref[...], a)
    eo = jnp.dot(eo, bd3_ref[...], preferred_element_type=jnp.float32) \
        + be3_ref[...]
    g = _prelu(jnp.dot(comb, wg1_ref[...], preferred_element_type=jnp.float32)
               + bg1_ref[...], a)
    g = _prelu(jnp.dot(g, wg2_ref[...], preferred_element_type=jnp.float32)
               + bg2_ref[...], a)
    gl = jnp.dot(g, wg3_ref[...], preferred_element_type=jnp.float32) \
        + bg3_ref[...]
    gl = gl - jnp.max(gl, axis=1, keepdims=True)
    egl = jnp.exp(gl)
    sm = egl / jnp.sum(egl, axis=1, keepdims=True)
    out = jnp.sum(eo * sm, axis=1, keepdims=True)
    out_ref[...] = out * wf_ref[...] + bf_ref[...]


def _head(solute, extra, We1fT, be1f, BD2, be2f, BD3, be3f,
          Wg1T, bg1f, Wg2T, bg2f, Wg3T, bg3f, Wf, bf2, a2):
    args = (solute, extra, We1fT, be1f, BD2, be2f, BD3, be3f,
            Wg1T, bg1f, Wg2T, bg2f, Wg3T, bg3f, Wf, bf2, a2)
    return pl.pallas_call(
        _head_body,
        grid=(1,),
        in_specs=[_full(x.shape) for x in args],
        out_specs=_full((1, 1)),
        out_shape=jax.ShapeDtypeStruct((1, 1), jnp.float32),
    )(*args)


def kernel(x, edge_attr, edge_index, extra_features, a_prelu, W_edge, W_eupd,
           W_node, b_node, We1, be1, We2, be2, We3, be3, Wg1, bg1, Wg2, bg2,
           Wg3, bg3, Wf, bf):
    src2 = edge_index[0].astype(jnp.int32).reshape(E // GR, GR)
    dst2 = edge_index[1].astype(jnp.int32).reshape(E // GR, GR)
    rev2 = (jnp.arange(E, dtype=jnp.int32) ^ 1).reshape(E // GR, GR)
    a2 = jnp.reshape(a_prelu.astype(jnp.float32), (1, 1))
    zeros_nd = jnp.zeros((N, D), jnp.float32)

    WxT = W_edge[:, :D].T
    WeT = W_edge[:, D:].T
    WuT = W_eupd.T
    Wn1T = W_node[:, :D].T
    Wn2T = W_node[:, D:].T
    b2 = b_node.reshape(1, D)

    NE = We1.shape[0]
    We1fT = We1.reshape(NE * 128, D + 16).T
    be1f = be1.reshape(1, NE * 128)
    BD2 = jax.scipy.linalg.block_diag(*[We2[e].T for e in range(NE)])
    be2f = be2.reshape(1, NE * 128)
    BD3 = jax.scipy.linalg.block_diag(*[We3[e].T for e in range(NE)])
    be3f = be3.reshape(1, NE)
    bf2 = bf.reshape(1, 1)

    xp, xn = _node_pre(x, WxT, Wn1T)
    g0 = _sc_gather_spm(xp, src2)
    h0 = _h0(g0, edge_attr, WeT, a2)
    H0 = _sc_swap_pairs(h0, rev2)

    g = _sc_round(h0, dst2, zeros_nd)
    H1 = _round_tc(g, h0, H0, WuT, a2)
    g = _sc_round(H1, src2, zeros_nd)
    h2 = _round_tc(g, H1, h0, WuT, a2)
    g = _sc_round(h2, dst2, zeros_nd)
    H3 = _round_tc(g, h2, H0, WuT, a2)

    ms0, ms1 = _sc_segsum(H3, src2, zeros_nd)
    solute = _node_out(xn, ms0, ms1, Wn2T, b2, a2)

    return _head(solute, extra_features, We1fT, be1f, BD2, be2f, BD3, be3f,
                 Wg1.T, bg1.reshape(1, 128), Wg2.T, bg2.reshape(1, 128),
                 Wg3.T, bg3.reshape(1, NE), Wf, bf2, a2)


# speedup vs baseline: 3.4441x; 1.0276x over previous
import functools

import jax
import jax.numpy as jnp
from jax import lax
from jax.experimental import pallas as pl
from jax.experimental.pallas import tpu as pltpu
from jax.experimental.pallas import tpu_sc as plsc

N = 10000
E = 320000
D = 128
NC = 2
NS = 16
NW = NC * NS

GR = 128
NG = E // GR
NSL = -(-NG // NS)
NWL = -(-NG // NW)
ZCH = 80
NZC = N // ZCH


def _prelu(v, a):
    return jnp.where(v >= 0, v, a * v)


_MESH = plsc.VectorSubcoreMesh(core_axis_name="c", subcore_axis_name="s")

_SC_SCRATCH = [
    pltpu.VMEM((1, 128), jnp.int32),
    pltpu.VMEM((1, 128), jnp.int32),
    pltpu.VMEM((GR, D), jnp.float32),
    pltpu.VMEM((GR, D), jnp.float32),
    pltpu.SemaphoreType.DMA,
    pltpu.SemaphoreType.DMA,
    pltpu.SemaphoreType.DMA,
    pltpu.SemaphoreType.DMA,
    pltpu.SemaphoreType.DMA,
    pltpu.SemaphoreType.DMA,
]


def _fill_table(src_hbm, table, sid):
    @pl.loop(0, -(-NZC // NS))
    def _z(i):
        tz = sid + i * NS

        @pl.when(tz < NZC)
        def _():
            r0 = tz * ZCH
            pltpu.sync_copy(src_hbm.at[pl.ds(r0, ZCH)], table.at[pl.ds(r0, ZCH)])


@functools.partial(
    pl.kernel,
    out_type=jax.ShapeDtypeStruct((E, D), jnp.float32),
    mesh=_MESH,
    scratch_types=_SC_SCRATCH + [pltpu.VMEM_SHARED((N, D), jnp.float32)],
)
def _sc_round(h_hbm, idx2_hbm, zeros_hbm, out_hbm,
              ia0, ia1, da0, da1, sia0, sia1, sda0, sda1, sst0, sst1, table):
    cid = lax.axis_index("c")
    sid = lax.axis_index("s")
    wid = cid * NS + sid
    ias = (ia0, ia1)
    das = (da0, da1)
    sias = (sia0, sia1)
    sdas = (sda0, sda1)
    ssts = (sst0, sst1)

    for b in range(2):
        t = sid + b * NS
        pltpu.async_copy(idx2_hbm.at[pl.ds(t, 1)], ias[b], sias[b])
        pltpu.async_copy(h_hbm.at[pl.ds(t * GR, GR)], das[b], sdas[b])
    _fill_table(zeros_hbm, table, sid)
    plsc.subcore_barrier()

    @pl.loop(0, -(-NSL // 2))
    def _sc(i2):
        for b in range(2):
            i = i2 * 2 + b
            t = sid + i * NS

            @pl.when(t < NG)
            def _():
                pltpu.make_async_copy(
                    idx2_hbm.at[pl.ds(t, 1)], ias[b], sias[b]).wait()
                pltpu.make_async_copy(
                    h_hbm.at[pl.ds(t * GR, GR)], das[b], sdas[b]).wait()
                pltpu.sync_copy(das[b], table.at[ias[b].at[0]], add=True)
                tn = sid + (i + 2) * NS

                @pl.when(tn < NG)
                def _():
                    pltpu.async_copy(idx2_hbm.at[pl.ds(tn, 1)], ias[b], sias[b])
                    pltpu.async_copy(
                        h_hbm.at[pl.ds(tn * GR, GR)], das[b], sdas[b])

    plsc.subcore_barrier()

    for b in range(2):
        t = wid + b * NW
        pltpu.async_copy(idx2_hbm.at[pl.ds(t, 1)], ias[b], sias[b])

    @pl.loop(0, -(-NWL // 2))
    def _ga(i2):
        for b in range(2):
            i = i2 * 2 + b
            t = wid + i * NW

            @pl.when(t < NG)
            def _():
                @pl.when(i >= 2)
                def _():
                    tprev = wid + (i - 2) * NW
                    pltpu.make_async_copy(
                        das[b], out_hbm.at[pl.ds(tprev * GR, GR)],
                        ssts[b]).wait()

                pltpu.make_async_copy(
                    idx2_hbm.at[pl.ds(t, 1)], ias[b], sias[b]).wait()
                pltpu.sync_copy(table.at[ias[b].at[0]], das[b])
                pltpu.async_copy(das[b], out_hbm.at[pl.ds(t * GR, GR)], ssts[b])
                tn = wid + (i + 2) * NW

                @pl.when(tn < NG)
                def _():
                    pltpu.async_copy(idx2_hbm.at[pl.ds(tn, 1)], ias[b], sias[b])

    imax = (NG - 1 - wid) // NW
    for b in range(2):
        ib = imax - lax.rem(imax - b, 2)
        tb = wid + ib * NW
        pltpu.make_async_copy(das[b], out_hbm.at[pl.ds(tb * GR, GR)],
                              ssts[b]).wait()


@functools.partial(
    pl.kernel,
    out_type=jax.ShapeDtypeStruct((E, D), jnp.float32),
    mesh=_MESH,
    scratch_types=_SC_SCRATCH + [pltpu.VMEM_SHARED((N, D), jnp.float32)],
)
def _sc_gather_spm(xp_hbm, idx2_hbm, out_hbm,
                   ia0, ia1, da0, da1, sia0, sia1, sda0, sda1, sst0, sst1,
                   table):
    cid = lax.axis_index("c")
    sid = lax.axis_index("s")
    wid = cid * NS + sid
    ias = (ia0, ia1)
    das = (da0, da1)
    sias = (sia0, sia1)
    ssts = (sst0, sst1)

    _fill_table(xp_hbm, table, sid)
    plsc.subcore_barrier()

    for b in range(2):
        t = wid + b * NW
        pltpu.async_copy(idx2_hbm.at[pl.ds(t, 1)], ias[b], sias[b])

    @pl.loop(0, -(-NWL // 2))
    def _ga(i2):
        for b in range(2):
            i = i2 * 2 + b
            t = wid + i * NW

            @pl.when(t < NG)
            def _():
                @pl.when(i >= 2)
                def _():
                    tprev = wid + (i - 2) * NW
                    pltpu.make_async_copy(
                        das[b], out_hbm.at[pl.ds(tprev * GR, GR)],
                        ssts[b]).wait()

                pltpu.make_async_copy(
                    idx2_hbm.at[pl.ds(t, 1)], ias[b], sias[b]).wait()
                pltpu.sync_copy(table.at[ias[b].at[0]], das[b])
                pltpu.async_copy(das[b], out_hbm.at[pl.ds(t * GR, GR)], ssts[b])
                tn = wid + (i + 2) * NW

                @pl.when(tn < NG)
                def _():
                    pltpu.async_copy(idx2_hbm.at[pl.ds(tn, 1)], ias[b], sias[b])

    imax = (NG - 1 - wid) // NW
    for b in range(2):
        ib = imax - lax.rem(imax - b, 2)
        tb = wid + ib * NW
        pltpu.make_async_copy(das[b], out_hbm.at[pl.ds(tb * GR, GR)],
                              ssts[b]).wait()


@functools.partial(
    pl.kernel,
    out_type=[jax.ShapeDtypeStruct((E, D), jnp.float32),
              jax.ShapeDtypeStruct((E, D), jnp.float32)],
    mesh=_MESH,
    scratch_types=_SC_SCRATCH + [
        pltpu.VMEM((1, 128), jnp.int32),
        pltpu.VMEM((1, 128), jnp.int32),
        pltpu.SemaphoreType.DMA,
        pltpu.SemaphoreType.DMA,
        pltpu.VMEM_SHARED((N, D), jnp.float32),
    ],
)
def _sc_round1(h_hbm, idx2_hbm, rev2_hbm, zeros_hbm, out_hbm, swap_hbm,
               ia0, ia1, da0, da1, sia0, sia1, sda0, sda1, sst0, sst1,
               ib0, ib1, sib0, sib1, table):
    cid = lax.axis_index("c")
    sid = lax.axis_index("s")
    wid = cid * NS + sid
    ias = (ia0, ia1)
    das = (da0, da1)
    sias = (sia0, sia1)
    sdas = (sda0, sda1)
    ssts = (sst0, sst1)
    ibs = (ib0, ib1)
    sibs = (sib0, sib1)

    for b in range(2):
        t = sid + b * NS
        pltpu.async_copy(idx2_hbm.at[pl.ds(t, 1)], ias[b], sias[b])
        pltpu.async_copy(h_hbm.at[pl.ds(t * GR, GR)], das[b], sdas[b])

        @pl.when(cid == b)
        def _():
            pltpu.async_copy(rev2_hbm.at[pl.ds(t, 1)], ibs[b], sibs[b])

    _fill_table(zeros_hbm, table, sid)
    plsc.subcore_barrier()

    @pl.loop(0, -(-NSL // 2))
    def _sc(i2):
        for b in range(2):
            i = i2 * 2 + b
            t = sid + i * NS

            @pl.when(t < NG)
            def _():
                pltpu.make_async_copy(
                    idx2_hbm.at[pl.ds(t, 1)], ias[b], sias[b]).wait()
                pltpu.make_async_copy(
                    h_hbm.at[pl.ds(t * GR, GR)], das[b], sdas[b]).wait()
                pltpu.sync_copy(das[b], table.at[ias[b].at[0]], add=True)
                tn = sid + (i + 2) * NS

                @pl.when(cid == b)
                def _():
                    pltpu.make_async_copy(
                        rev2_hbm.at[pl.ds(t, 1)], ibs[b], sibs[b]).wait()
                    pltpu.sync_copy(das[b], swap_hbm.at[ibs[b].at[0]])

                    @pl.when(tn < NG)
                    def _():
                        pltpu.async_copy(
                            rev2_hbm.at[pl.ds(tn, 1)], ibs[b], sibs[b])

                @pl.when(tn < NG)
                def _():
                    pltpu.async_copy(idx2_hbm.at[pl.ds(tn, 1)], ias[b], sias[b])
                    pltpu.async_copy(
                        h_hbm.at[pl.ds(tn * GR, GR)], das[b], sdas[b])

    plsc.subcore_barrier()

    for b in range(2):
        t = wid + b * NW
        pltpu.async_copy(idx2_hbm.at[pl.ds(t, 1)], ias[b], sias[b])

    @pl.loop(0, -(-NWL // 2))
    def _ga(i2):
        for b in range(2):
            i = i2 * 2 + b
            t = wid + i * NW

            @pl.when(t < NG)
            def _():
                @pl.when(i >= 2)
                def _():
                    tprev = wid + (i - 2) * NW
                    pltpu.make_async_copy(
                        das[b], out_hbm.at[pl.ds(tprev * GR, GR)],
                        ssts[b]).wait()

                pltpu.make_async_copy(
                    idx2_hbm.at[pl.ds(t, 1)], ias[b], sias[b]).wait()
                pltpu.sync_copy(table.at[ias[b].at[0]], das[b])
                pltpu.async_copy(das[b], out_hbm.at[pl.ds(t * GR, GR)], ssts[b])
                tn = wid + (i + 2) * NW

                @pl.when(tn < NG)
                def _():
                    pltpu.async_copy(idx2_hbm.at[pl.ds(tn, 1)], ias[b], sias[b])

    imax = (NG - 1 - wid) // NW
    for b in range(2):
        ib = imax - lax.rem(imax - b, 2)
        tb = wid + ib * NW
        pltpu.make_async_copy(das[b], out_hbm.at[pl.ds(tb * GR, GR)],
                              ssts[b]).wait()


@functools.partial(
    pl.kernel,
    out_type=[jax.ShapeDtypeStruct((N, D), jnp.float32),
              jax.ShapeDtypeStruct((N, D), jnp.float32)],
    mesh=_MESH,
    scratch_types=_SC_SCRATCH + [pltpu.VMEM_SHARED((N, D), jnp.float32)],
)
def _sc_segsum(h_hbm, idx2_hbm, zeros_hbm, out0_hbm, out1_hbm,
               ia0, ia1, da0, da1, sia0, sia1, sda0, sda1, sst0, sst1, table):
    cid = lax.axis_index("c")
    sid = lax.axis_index("s")
    ias = (ia0, ia1)
    das = (da0, da1)
    sias = (sia0, sia1)
    sdas = (sda0, sda1)

    for b in range(2):
        t = sid + (2 * b + cid) * NS
        pltpu.async_copy(idx2_hbm.at[pl.ds(t, 1)], ias[b], sias[b])
        pltpu.async_copy(h_hbm.at[pl.ds(t * GR, GR)], das[b], sdas[b])
    _fill_table(zeros_hbm, table, sid)
    plsc.subcore_barrier()

    NHL = -(-NSL // 2)
    @pl.loop(0, -(-NHL // 2))
    def _sc(j2):
        for b in range(2):
            j = j2 * 2 + b
            t = sid + (2 * j + cid) * NS

            @pl.when(t < NG)
            def _():
                pltpu.make_async_copy(
                    idx2_hbm.at[pl.ds(t, 1)], ias[b], sias[b]).wait()
                pltpu.make_async_copy(
                    h_hbm.at[pl.ds(t * GR, GR)], das[b], sdas[b]).wait()
                pltpu.sync_copy(das[b], table.at[ias[b].at[0]], add=True)
                tn = sid + (2 * (j + 2) + cid) * NS

                @pl.when(tn < NG)
                def _():
                    pltpu.async_copy(idx2_hbm.at[pl.ds(tn, 1)], ias[b], sias[b])
                    pltpu.async_copy(
                        h_hbm.at[pl.ds(tn * GR, GR)], das[b], sdas[b])

    plsc.subcore_barrier()

    @pl.loop(0, -(-NZC // NS))
    def _chunks(i):
        t = sid + i * NS

        @pl.when(t < NZC)
        def _():
            r0 = t * ZCH
            pltpu.sync_copy(table.at[pl.ds(r0, ZCH)], da0.at[pl.ds(0, ZCH)])

            @pl.when(cid == 0)
            def _():
                pltpu.sync_copy(da0.at[pl.ds(0, ZCH)],
                                out0_hbm.at[pl.ds(r0, ZCH)])

            @pl.when(cid == 1)
            def _():
                pltpu.sync_copy(da0.at[pl.ds(0, ZCH)],
                                out1_hbm.at[pl.ds(r0, ZCH)])


def _full(shape):
    return pl.BlockSpec(shape, lambda *_: tuple(0 for _ in shape))


def _node_pre_body(x_ref, wx_ref, wn_ref, xp_ref, xn_ref):
    xb = x_ref[...]
    xp_ref[...] = jnp.dot(xb, wx_ref[...], preferred_element_type=jnp.float32)
    xn_ref[...] = jnp.dot(xb, wn_ref[...], preferred_element_type=jnp.float32)


def _node_pre(x, WxT, Wn1T):
    bn = 1000
    return pl.pallas_call(
        _node_pre_body,
        grid=(N // bn,),
        in_specs=[
            pl.BlockSpec((bn, D), lambda i: (i, 0)),
            _full((D, D)),
            _full((D, D)),
        ],
        out_specs=[
            pl.BlockSpec((bn, D), lambda i: (i, 0)),
            pl.BlockSpec((bn, D), lambda i: (i, 0)),
        ],
        out_shape=[
            jax.ShapeDtypeStruct((N, D), jnp.float32),
            jax.ShapeDtypeStruct((N, D), jnp.float32),
        ],
    )(x, WxT, Wn1T)


def _h0_body(g0_ref, ea_ref, we_ref, a_ref, out_ref):
    a = a_ref[0, 0]
    y = g0_ref[...] + jnp.dot(ea_ref[...], we_ref[...],
                              preferred_element_type=jnp.float32)
    out_ref[...] = _prelu(y, a)


def _h0(g0, edge_attr, WeT, a2):
    be = 3200
    return pl.pallas_call(
        _h0_body,
        grid=(E // be,),
        in_specs=[
            pl.BlockSpec((be, D), lambda i: (i, 0)),
            pl.BlockSpec((be, 16), lambda i: (i, 0)),
            _full((16, D)),
            _full((1, 1)),
        ],
        out_specs=pl.BlockSpec((be, D), lambda i: (i, 0)),
        out_shape=jax.ShapeDtypeStruct((E, D), jnp.float32),
    )(g0, edge_attr, WeT, a2)


def _round_body(g_ref, h_ref, h0_ref, wu_ref, a_ref, out_ref):
    a = a_ref[0, 0]
    m = g_ref[...] - h_ref[...]
    y = jnp.dot(m, wu_ref[...], preferred_element_type=jnp.float32)
    out_ref[...] = _prelu(y + h0_ref[...], a)


def _round_tc(g, h, h0sel, WuT, a2):
    be = 3200
    return pl.pallas_call(
        _round_body,
        grid=(E // be,),
        in_specs=[
            pl.BlockSpec((be, D), lambda i: (i, 0)),
            pl.BlockSpec((be, D), lambda i: (i, 0)),
            pl.BlockSpec((be, D), lambda i: (i, 0)),
            _full((D, D)),
            _full((1, 1)),
        ],
        out_specs=pl.BlockSpec((be, D), lambda i: (i, 0)),
        out_shape=jax.ShapeDtypeStruct((E, D), jnp.float32),
    )(g, h, h0sel, WuT, a2)


def _node_out_body(xn_ref, ms0_ref, ms1_ref, wn2_ref, b_ref, a_ref, out_ref):
    a = a_ref[0, 0]
    ms = ms0_ref[...] + ms1_ref[...]
    hn = _prelu(
        xn_ref[...]
        + jnp.dot(ms, wn2_ref[...], preferred_element_type=jnp.float32)
        + b_ref[...],
        a,
    )
    part = jnp.sum(hn, axis=0, keepdims=True)

    @pl.when(pl.program_id(0) == 0)
    def _():
        out_ref[...] = jnp.zeros_like(out_ref)

    out_ref[...] += part


def _node_out(xn, ms0, ms1, Wn2T, b2, a2):
    bn = 1000
    return pl.pallas_call(
        _node_out_body,
        grid=(N // bn,),
        in_specs=[
            pl.BlockSpec((bn, D), lambda i: (i, 0)),
            pl.BlockSpec((bn, D), lambda i: (i, 0)),
            pl.BlockSpec((bn, D), lambda i: (i, 0)),
            _full((D, D)),
            _full((1, D)),
            _full((1, 1)),
        ],
        out_specs=pl.BlockSpec((1, D), lambda i: (0, 0)),
        out_shape=jax.ShapeDtypeStruct((1, D), jnp.float32),
    )(xn, ms0, ms1, Wn2T, b2, a2)


def _head_body(sol_ref, ex_ref, we1_ref, be1_ref, bd2_ref, be2_ref, bd3_ref,
               be3_ref, wg1_ref, bg1_ref, wg2_ref, bg2_ref, wg3_ref, bg3_ref,
               wf_ref, bf_ref, a_ref, out_ref):
    a = a_ref[0, 0]
    comb = jnp.concatenate([sol_ref[...], ex_ref[...]], axis=1)
    eo = _prelu(jnp.dot(comb, we1_ref[...], preferred_element_type=jnp.float32)
                + be1_ref[...], a)
    eo = _prelu(jnp.dot(eo, bd2_ref[...], preferred_element_type=jnp.float32)
                + be2_ref[...], a)
    eo = jnp.dot(eo, bd3_ref[...], preferred_element_type=jnp.float32) \
        + be3_ref[...]
    g = _prelu(jnp.dot(comb, wg1_ref[...], preferred_element_type=jnp.float32)
               + bg1_ref[...], a)
    g = _prelu(jnp.dot(g, wg2_ref[...], preferred_element_type=jnp.float32)
               + bg2_ref[...], a)
    gl = jnp.dot(g, wg3_ref[...], preferred_element_type=jnp.float32) \
        + bg3_ref[...]
    gl = gl - jnp.max(gl, axis=1, keepdims=True)
    egl = jnp.exp(gl)
    sm = egl / jnp.sum(egl, axis=1, keepdims=True)
    out = jnp.sum(eo * sm, axis=1, keepdims=True)
    out_ref[...] = out * wf_ref[...] + bf_ref[...]


def _head(solute, extra, We1fT, be1f, BD2, be2f, BD3, be3f,
          Wg1T, bg1f, Wg2T, bg2f, Wg3T, bg3f, Wf, bf2, a2):
    args = (solute, extra, We1fT, be1f, BD2, be2f, BD3, be3f,
            Wg1T, bg1f, Wg2T, bg2f, Wg3T, bg3f, Wf, bf2, a2)
    return pl.pallas_call(
        _head_body,
        grid=(1,),
        in_specs=[_full(x.shape) for x in args],
        out_specs=_full((1, 1)),
        out_shape=jax.ShapeDtypeStruct((1, 1), jnp.float32),
    )(*args)


def kernel(x, edge_attr, edge_index, extra_features, a_prelu, W_edge, W_eupd,
           W_node, b_node, We1, be1, We2, be2, We3, be3, Wg1, bg1, Wg2, bg2,
           Wg3, bg3, Wf, bf):
    src2 = edge_index[0].astype(jnp.int32).reshape(E // GR, GR)
    dst2 = edge_index[1].astype(jnp.int32).reshape(E // GR, GR)
    rev2 = (jnp.arange(E, dtype=jnp.int32) ^ 1).reshape(E // GR, GR)
    a2 = jnp.reshape(a_prelu.astype(jnp.float32), (1, 1))
    zeros_nd = jnp.zeros((N, D), jnp.float32)

    WxT = W_edge[:, :D].T
    WeT = W_edge[:, D:].T
    WuT = W_eupd.T
    Wn1T = W_node[:, :D].T
    Wn2T = W_node[:, D:].T
    b2 = b_node.reshape(1, D)

    NE = We1.shape[0]
    We1fT = We1.reshape(NE * 128, D + 16).T
    be1f = be1.reshape(1, NE * 128)
    BD2 = jax.scipy.linalg.block_diag(*[We2[e].T for e in range(NE)])
    be2f = be2.reshape(1, NE * 128)
    BD3 = jax.scipy.linalg.block_diag(*[We3[e].T for e in range(NE)])
    be3f = be3.reshape(1, NE)
    bf2 = bf.reshape(1, 1)

    xp, xn = _node_pre(x, WxT, Wn1T)
    g0 = _sc_gather_spm(xp, src2)
    h0 = _h0(g0, edge_attr, WeT, a2)

    g, H0 = _sc_round1(h0, dst2, rev2, zeros_nd)
    H1 = _round_tc(g, h0, H0, WuT, a2)
    g = _sc_round(H1, src2, zeros_nd)
    h2 = _round_tc(g, H1, h0, WuT, a2)
    g = _sc_round(h2, dst2, zeros_nd)
    H3 = _round_tc(g, h2, H0, WuT, a2)

    ms0, ms1 = _sc_segsum(H3, src2, zeros_nd)
    solute = _node_out(xn, ms0, ms1, Wn2T, b2, a2)

    return _head(solute, extra_features, We1fT, be1f, BD2, be2f, BD3, be3f,
                 Wg1.T, bg1.reshape(1, 128), Wg2.T, bg2.reshape(1, 128),
                 Wg3.T, bg3.reshape(1, NE), Wf, bf2, a2)


# speedup vs baseline: 3.4457x; 1.0005x over previous
import functools

import jax
import jax.numpy as jnp
from jax import lax
from jax.experimental import pallas as pl
from jax.experimental.pallas import tpu as pltpu
from jax.experimental.pallas import tpu_sc as plsc

N = 10000
E = 320000
D = 128
NC = 2
NS = 16
NW = NC * NS

GR = 128
NG = E // GR
NSL = -(-NG // NS)
NWL = -(-NG // NW)
ZCH = 80
NZC = N // ZCH


def _prelu(v, a):
    return jnp.where(v >= 0, v, a * v)


_MESH = plsc.VectorSubcoreMesh(core_axis_name="c", subcore_axis_name="s")

_SC_SCRATCH = [
    pltpu.VMEM((1, GR), jnp.int32),
    pltpu.VMEM((1, GR), jnp.int32),
    pltpu.VMEM((GR, D), jnp.float32),
    pltpu.VMEM((GR, D), jnp.float32),
    pltpu.SemaphoreType.DMA,
    pltpu.SemaphoreType.DMA,
    pltpu.SemaphoreType.DMA,
    pltpu.SemaphoreType.DMA,
    pltpu.SemaphoreType.DMA,
    pltpu.SemaphoreType.DMA,
]


def _fill_table(src_hbm, table, sid):
    @pl.loop(0, -(-NZC // NS))
    def _z(i):
        tz = sid + i * NS

        @pl.when(tz < NZC)
        def _():
            r0 = tz * ZCH
            pltpu.sync_copy(src_hbm.at[pl.ds(r0, ZCH)], table.at[pl.ds(r0, ZCH)])


@functools.partial(
    pl.kernel,
    out_type=jax.ShapeDtypeStruct((E, D), jnp.float32),
    mesh=_MESH,
    scratch_types=_SC_SCRATCH + [pltpu.VMEM_SHARED((N, D), jnp.float32)],
)
def _sc_round(h_hbm, idx2_hbm, zeros_hbm, out_hbm,
              ia0, ia1, da0, da1, sia0, sia1, sda0, sda1, sst0, sst1, table):
    cid = lax.axis_index("c")
    sid = lax.axis_index("s")
    wid = cid * NS + sid
    ias = (ia0, ia1)
    das = (da0, da1)
    sias = (sia0, sia1)
    sdas = (sda0, sda1)
    ssts = (sst0, sst1)

    for b in range(2):
        t = sid + b * NS
        pltpu.async_copy(idx2_hbm.at[pl.ds(t, 1)], ias[b], sias[b])
        pltpu.async_copy(h_hbm.at[pl.ds(t * GR, GR)], das[b], sdas[b])
    _fill_table(zeros_hbm, table, sid)
    plsc.subcore_barrier()

    @pl.loop(0, -(-NSL // 2))
    def _sc(i2):
        for b in range(2):
            i = i2 * 2 + b
            t = sid + i * NS

            @pl.when(t < NG)
            def _():
                pltpu.make_async_copy(
                    idx2_hbm.at[pl.ds(t, 1)], ias[b], sias[b]).wait()
                pltpu.make_async_copy(
                    h_hbm.at[pl.ds(t * GR, GR)], das[b], sdas[b]).wait()
                pltpu.sync_copy(das[b], table.at[ias[b].at[0]], add=True)
                tn = sid + (i + 2) * NS

                @pl.when(tn < NG)
                def _():
                    pltpu.async_copy(idx2_hbm.at[pl.ds(tn, 1)], ias[b], sias[b])
                    pltpu.async_copy(
                        h_hbm.at[pl.ds(tn * GR, GR)], das[b], sdas[b])

    plsc.subcore_barrier()

    for b in range(2):
        t = wid + b * NW
        pltpu.async_copy(idx2_hbm.at[pl.ds(t, 1)], ias[b], sias[b])

    @pl.loop(0, -(-NWL // 2))
    def _ga(i2):
        for b in range(2):
            i = i2 * 2 + b
            t = wid + i * NW

            @pl.when(t < NG)
            def _():
                @pl.when(i >= 2)
                def _():
                    tprev = wid + (i - 2) * NW
                    pltpu.make_async_copy(
                        das[b], out_hbm.at[pl.ds(tprev * GR, GR)],
                        ssts[b]).wait()

                pltpu.make_async_copy(
                    idx2_hbm.at[pl.ds(t, 1)], ias[b], sias[b]).wait()
                pltpu.sync_copy(table.at[ias[b].at[0]], das[b])
                pltpu.async_copy(das[b], out_hbm.at[pl.ds(t * GR, GR)], ssts[b])
                tn = wid + (i + 2) * NW

                @pl.when(tn < NG)
                def _():
                    pltpu.async_copy(idx2_hbm.at[pl.ds(tn, 1)], ias[b], sias[b])

    imax = (NG - 1 - wid) // NW
    for b in range(2):
        ib = imax - lax.rem(imax - b, 2)
        tb = wid + ib * NW
        pltpu.make_async_copy(das[b], out_hbm.at[pl.ds(tb * GR, GR)],
                              ssts[b]).wait()


@functools.partial(
    pl.kernel,
    out_type=jax.ShapeDtypeStruct((E, D), jnp.float32),
    mesh=_MESH,
    scratch_types=_SC_SCRATCH + [pltpu.VMEM_SHARED((N, D), jnp.float32)],
)
def _sc_gather_spm(xp_hbm, idx2_hbm, out_hbm,
                   ia0, ia1, da0, da1, sia0, sia1, sda0, sda1, sst0, sst1,
                   table):
    cid = lax.axis_index("c")
    sid = lax.axis_index("s")
    wid = cid * NS + sid
    ias = (ia0, ia1)
    das = (da0, da1)
    sias = (sia0, sia1)
    ssts = (sst0, sst1)

    _fill_table(xp_hbm, table, sid)
    plsc.subcore_barrier()

    for b in range(2):
        t = wid + b * NW
        pltpu.async_copy(idx2_hbm.at[pl.ds(t, 1)], ias[b], sias[b])

    @pl.loop(0, -(-NWL // 2))
    def _ga(i2):
        for b in range(2):
            i = i2 * 2 + b
            t = wid + i * NW

            @pl.when(t < NG)
            def _():
                @pl.when(i >= 2)
                def _():
                    tprev = wid + (i - 2) * NW
                    pltpu.make_async_copy(
                        das[b], out_hbm.at[pl.ds(tprev * GR, GR)],
                        ssts[b]).wait()

                pltpu.make_async_copy(
                    idx2_hbm.at[pl.ds(t, 1)], ias[b], sias[b]).wait()
                pltpu.sync_copy(table.at[ias[b].at[0]], das[b])
                pltpu.async_copy(das[b], out_hbm.at[pl.ds(t * GR, GR)], ssts[b])
                tn = wid + (i + 2) * NW

                @pl.when(tn < NG)
                def _():
                    pltpu.async_copy(idx2_hbm.at[pl.ds(tn, 1)], ias[b], sias[b])

    imax = (NG - 1 - wid) // NW
    for b in range(2):
        ib = imax - lax.rem(imax - b, 2)
        tb = wid + ib * NW
        pltpu.make_async_copy(das[b], out_hbm.at[pl.ds(tb * GR, GR)],
                              ssts[b]).wait()


@functools.partial(
    pl.kernel,
    out_type=[jax.ShapeDtypeStruct((E, D), jnp.float32),
              jax.ShapeDtypeStruct((E, D), jnp.float32)],
    mesh=_MESH,
    scratch_types=_SC_SCRATCH + [
        pltpu.VMEM((1, GR), jnp.int32),
        pltpu.VMEM((1, GR), jnp.int32),
        pltpu.SemaphoreType.DMA,
        pltpu.SemaphoreType.DMA,
        pltpu.VMEM_SHARED((N, D), jnp.float32),
    ],
)
def _sc_round1(h_hbm, idx2_hbm, rev2_hbm, zeros_hbm, out_hbm, swap_hbm,
               ia0, ia1, da0, da1, sia0, sia1, sda0, sda1, sst0, sst1,
               ib0, ib1, sib0, sib1, table):
    cid = lax.axis_index("c")
    sid = lax.axis_index("s")
    wid = cid * NS + sid
    ias = (ia0, ia1)
    das = (da0, da1)
    sias = (sia0, sia1)
    sdas = (sda0, sda1)
    ssts = (sst0, sst1)
    ibs = (ib0, ib1)
    sibs = (sib0, sib1)

    for b in range(2):
        t = sid + b * NS
        pltpu.async_copy(idx2_hbm.at[pl.ds(t, 1)], ias[b], sias[b])
        pltpu.async_copy(h_hbm.at[pl.ds(t * GR, GR)], das[b], sdas[b])

        @pl.when(cid == b)
        def _():
            pltpu.async_copy(rev2_hbm.at[pl.ds(t, 1)], ibs[b], sibs[b])

    _fill_table(zeros_hbm, table, sid)
    plsc.subcore_barrier()

    @pl.loop(0, -(-NSL // 2))
    def _sc(i2):
        for b in range(2):
            i = i2 * 2 + b
            t = sid + i * NS

            @pl.when(t < NG)
            def _():
                pltpu.make_async_copy(
                    idx2_hbm.at[pl.ds(t, 1)], ias[b], sias[b]).wait()
                pltpu.make_async_copy(
                    h_hbm.at[pl.ds(t * GR, GR)], das[b], sdas[b]).wait()
                pltpu.sync_copy(das[b], table.at[ias[b].at[0]], add=True)
                tn = sid + (i + 2) * NS

                @pl.when(cid == b)
                def _():
                    pltpu.make_async_copy(
                        rev2_hbm.at[pl.ds(t, 1)], ibs[b], sibs[b]).wait()
                    pltpu.sync_copy(das[b], swap_hbm.at[ibs[b].at[0]])

                    @pl.when(tn < NG)
                    def _():
                        pltpu.async_copy(
                            rev2_hbm.at[pl.ds(tn, 1)], ibs[b], sibs[b])

                @pl.when(tn < NG)
                def _():
                    pltpu.async_copy(idx2_hbm.at[pl.ds(tn, 1)], ias[b], sias[b])
                    pltpu.async_copy(
                        h_hbm.at[pl.ds(tn * GR, GR)], das[b], sdas[b])

    plsc.subcore_barrier()

    for b in range(2):
        t = wid + b * NW
        pltpu.async_copy(idx2_hbm.at[pl.ds(t, 1)], ias[b], sias[b])

    @pl.loop(0, -(-NWL // 2))
    def _ga(i2):
        for b in range(2):
            i = i2 * 2 + b
            t = wid + i * NW

            @pl.when(t < NG)
            def _():
                @pl.when(i >= 2)
                def _():
                    tprev = wid + (i - 2) * NW
                    pltpu.make_async_copy(
                        das[b], out_hbm.at[pl.ds(tprev * GR, GR)],
                        ssts[b]).wait()

                pltpu.make_async_copy(
                    idx2_hbm.at[pl.ds(t, 1)], ias[b], sias[b]).wait()
                pltpu.sync_copy(table.at[ias[b].at[0]], das[b])
                pltpu.async_copy(das[b], out_hbm.at[pl.ds(t * GR, GR)], ssts[b])
                tn = wid + (i + 2) * NW

                @pl.when(tn < NG)
                def _():
                    pltpu.async_copy(idx2_hbm.at[pl.ds(tn, 1)], ias[b], sias[b])

    imax = (NG - 1 - wid) // NW
    for b in range(2):
        ib = imax - lax.rem(imax - b, 2)
        tb = wid + ib * NW
        pltpu.make_async_copy(das[b], out_hbm.at[pl.ds(tb * GR, GR)],
                              ssts[b]).wait()


@functools.partial(
    pl.kernel,
    out_type=[jax.ShapeDtypeStruct((N, D), jnp.float32),
              jax.ShapeDtypeStruct((N, D), jnp.float32)],
    mesh=_MESH,
    scratch_types=_SC_SCRATCH + [pltpu.VMEM_SHARED((N, D), jnp.float32)],
)
def _sc_segsum(h_hbm, idx2_hbm, zeros_hbm, out0_hbm, out1_hbm,
               ia0, ia1, da0, da1, sia0, sia1, sda0, sda1, sst0, sst1, table):
    cid = lax.axis_index("c")
    sid = lax.axis_index("s")
    ias = (ia0, ia1)
    das = (da0, da1)
    sias = (sia0, sia1)
    sdas = (sda0, sda1)

    for b in range(2):
        t = sid + (2 * b + cid) * NS
        pltpu.async_copy(idx2_hbm.at[pl.ds(t, 1)], ias[b], sias[b])
        pltpu.async_copy(h_hbm.at[pl.ds(t * GR, GR)], das[b], sdas[b])
    _fill_table(zeros_hbm, table, sid)
    plsc.subcore_barrier()

    NHL = -(-NSL // 2)
    @pl.loop(0, -(-NHL // 2))
    def _sc(j2):
        for b in range(2):
            j = j2 * 2 + b
            t = sid + (2 * j + cid) * NS

            @pl.when(t < NG)
            def _():
                pltpu.make_async_copy(
                    idx2_hbm.at[pl.ds(t, 1)], ias[b], sias[b]).wait()
                pltpu.make_async_copy(
                    h_hbm.at[pl.ds(t * GR, GR)], das[b], sdas[b]).wait()
                pltpu.sync_copy(das[b], table.at[ias[b].at[0]], add=True)
                tn = sid + (2 * (j + 2) + cid) * NS

                @pl.when(tn < NG)
                def _():
                    pltpu.async_copy(idx2_hbm.at[pl.ds(tn, 1)], ias[b], sias[b])
                    pltpu.async_copy(
                        h_hbm.at[pl.ds(tn * GR, GR)], das[b], sdas[b])

    plsc.subcore_barrier()

    @pl.loop(0, -(-NZC // NS))
    def _chunks(i):
        t = sid + i * NS

        @pl.when(t < NZC)
        def _():
            r0 = t * ZCH
            pltpu.sync_copy(table.at[pl.ds(r0, ZCH)], da0.at[pl.ds(0, ZCH)])

            @pl.when(cid == 0)
            def _():
                pltpu.sync_copy(da0.at[pl.ds(0, ZCH)],
                                out0_hbm.at[pl.ds(r0, ZCH)])

            @pl.when(cid == 1)
            def _():
                pltpu.sync_copy(da0.at[pl.ds(0, ZCH)],
                                out1_hbm.at[pl.ds(r0, ZCH)])


def _full(shape):
    return pl.BlockSpec(shape, lambda *_: tuple(0 for _ in shape))


def _node_pre_body(x_ref, wx_ref, wn_ref, xp_ref, xn_ref):
    xb = x_ref[...]
    xp_ref[...] = jnp.dot(xb, wx_ref[...], preferred_element_type=jnp.float32)
    xn_ref[...] = jnp.dot(xb, wn_ref[...], preferred_element_type=jnp.float32)


def _node_pre(x, WxT, Wn1T):
    bn = 1000
    return pl.pallas_call(
        _node_pre_body,
        grid=(N // bn,),
        in_specs=[
            pl.BlockSpec((bn, D), lambda i: (i, 0)),
            _full((D, D)),
            _full((D, D)),
        ],
        out_specs=[
            pl.BlockSpec((bn, D), lambda i: (i, 0)),
            pl.BlockSpec((bn, D), lambda i: (i, 0)),
        ],
        out_shape=[
            jax.ShapeDtypeStruct((N, D), jnp.float32),
            jax.ShapeDtypeStruct((N, D), jnp.float32),
        ],
    )(x, WxT, Wn1T)


def _h0_body(g0_ref, ea_ref, we_ref, a_ref, out_ref):
    a = a_ref[0, 0]
    y = g0_ref[...] + jnp.dot(ea_ref[...], we_ref[...],
                              preferred_element_type=jnp.float32)
    out_ref[...] = _prelu(y, a)


def _h0(g0, edge_attr, WeT, a2):
    be = 3200
    return pl.pallas_call(
        _h0_body,
        grid=(E // be,),
        in_specs=[
            pl.BlockSpec((be, D), lambda i: (i, 0)),
            pl.BlockSpec((be, 16), lambda i: (i, 0)),
            _full((16, D)),
            _full((1, 1)),
        ],
        out_specs=pl.BlockSpec((be, D), lambda i: (i, 0)),
        out_shape=jax.ShapeDtypeStruct((E, D), jnp.float32),
    )(g0, edge_attr, WeT, a2)


def _round_body(g_ref, h_ref, h0_ref, wu_ref, a_ref, out_ref):
    a = a_ref[0, 0]
    m = g_ref[...] - h_ref[...]
    y = jnp.dot(m, wu_ref[...], preferred_element_type=jnp.float32)
    out_ref[...] = _prelu(y + h0_ref[...], a)


def _round_tc(g, h, h0sel, WuT, a2):
    be = 3200
    return pl.pallas_call(
        _round_body,
        grid=(E // be,),
        in_specs=[
            pl.BlockSpec((be, D), lambda i: (i, 0)),
            pl.BlockSpec((be, D), lambda i: (i, 0)),
            pl.BlockSpec((be, D), lambda i: (i, 0)),
            _full((D, D)),
            _full((1, 1)),
        ],
        out_specs=pl.BlockSpec((be, D), lambda i: (i, 0)),
        out_shape=jax.ShapeDtypeStruct((E, D), jnp.float32),
    )(g, h, h0sel, WuT, a2)


def _node_out_body(xn_ref, ms0_ref, ms1_ref, wn2_ref, b_ref, a_ref, out_ref):
    a = a_ref[0, 0]
    ms = ms0_ref[...] + ms1_ref[...]
    hn = _prelu(
        xn_ref[...]
        + jnp.dot(ms, wn2_ref[...], preferred_element_type=jnp.float32)
        + b_ref[...],
        a,
    )
    part = jnp.sum(hn, axis=0, keepdims=True)

    @pl.when(pl.program_id(0) == 0)
    def _():
        out_ref[...] = jnp.zeros_like(out_ref)

    out_ref[...] += part


def _node_out(xn, ms0, ms1, Wn2T, b2, a2):
    bn = 1000
    return pl.pallas_call(
        _node_out_body,
        grid=(N // bn,),
        in_specs=[
            pl.BlockSpec((bn, D), lambda i: (i, 0)),
            pl.BlockSpec((bn, D), lambda i: (i, 0)),
            pl.BlockSpec((bn, D), lambda i: (i, 0)),
            _full((D, D)),
            _full((1, D)),
            _full((1, 1)),
        ],
        out_specs=pl.BlockSpec((1, D), lambda i: (0, 0)),
        out_shape=jax.ShapeDtypeStruct((1, D), jnp.float32),
    )(xn, ms0, ms1, Wn2T, b2, a2)


def _moe_head(solute, extra_features, a_prelu, We1, be1, We2, be2, We3, be3,
              Wg1, bg1, Wg2, bg2, Wg3, bg3, Wf, bf):
    def prelu(v):
        return jnp.where(v >= 0, v, a_prelu * v)
    comb = jnp.concatenate([solute, extra_features], axis=-1)
    eo = prelu(jnp.einsum('bd,ehd->beh', comb, We1) + be1)
    eo = prelu(jnp.einsum('beh,eoh->beo', eo, We2) + be2)
    eo = jnp.einsum('beh,eoh->beo', eo, We3) + be3
    g = prelu(comb @ Wg1.T + bg1)
    g = prelu(g @ Wg2.T + bg2)
    g = jax.nn.softmax(g @ Wg3.T + bg3, axis=1)
    out = jnp.sum(eo * g[..., None], axis=1).reshape(-1, 1)
    return out @ Wf.T + bf


def kernel(x, edge_attr, edge_index, extra_features, a_prelu, W_edge, W_eupd,
           W_node, b_node, We1, be1, We2, be2, We3, be3, Wg1, bg1, Wg2, bg2,
           Wg3, bg3, Wf, bf):
    src2 = edge_index[0].astype(jnp.int32).reshape(E // GR, GR)
    dst2 = edge_index[1].astype(jnp.int32).reshape(E // GR, GR)
    rev2 = (jnp.arange(E, dtype=jnp.int32) ^ 1).reshape(E // GR, GR)
    a2 = jnp.reshape(a_prelu.astype(jnp.float32), (1, 1))
    zeros_nd = jnp.zeros((N, D), jnp.float32)

    WxT = W_edge[:, :D].T
    WeT = W_edge[:, D:].T
    WuT = W_eupd.T
    Wn1T = W_node[:, :D].T
    Wn2T = W_node[:, D:].T
    b2 = b_node.reshape(1, D)

    xp, xn = _node_pre(x, WxT, Wn1T)
    g0 = _sc_gather_spm(xp, src2)
    h0 = _h0(g0, edge_attr, WeT, a2)

    g, H0 = _sc_round1(h0, dst2, rev2, zeros_nd)
    H1 = _round_tc(g, h0, H0, WuT, a2)
    g = _sc_round(H1, src2, zeros_nd)
    h2 = _round_tc(g, H1, h0, WuT, a2)
    g = _sc_round(h2, dst2, zeros_nd)
    H3 = _round_tc(g, h2, H0, WuT, a2)

    ms0, ms1 = _sc_segsum(H3, src2, zeros_nd)
    solute = _node_out(xn, ms0, ms1, Wn2T, b2, a2)

    return _moe_head(solute, extra_features, a_prelu, We1, be1, We2, be2,
                     We3, be3, Wg1, bg1, Wg2, bg2, Wg3, bg3, Wf, bf)
